# baseline (speedup 1.0000x reference)
import functools

import jax
import jax.numpy as jnp
from jax.experimental import pallas as pl
from jax.experimental.pallas import tpu as pltpu

_EMBED = 64
_HEADS = 4
_MODEL_CAT_RATE = 0.02
_ID_CAT_RATE = 0.36
_BM = 512
_F32 = jnp.float32


def _dot(a, b):
    return jnp.dot(a, b, preferred_element_type=_F32)


def _row_normalize(z):
    n = jnp.sqrt(jnp.sum(z * z, axis=1, keepdims=True))
    return z / jnp.maximum(n, 1e-12)


def _id_update(emb, a, b, w_sum):
    return emb + _ID_CAT_RATE * _row_normalize(_dot(0.5 * (a + b), w_sum))


_MK1_BM = 256


def _mk1_body(imf_ref, wi_ref, bi_ref, tf_ref, wt_ref, bt_ref,
              g_img_ui_ref, g_txt_ui_ref, g_img_iu_ref, g_txt_iu_ref,
              item_emb_ref, user_emb_ref,
              o_imgf, o_txtf, o_img_uid, o_txt_uid, o_img_iid, o_txt_iid):
    i = pl.program_id(0)
    nb = g_img_ui_ref.shape[1] // g_img_ui_ref.shape[0]

    @pl.when(i < nb)
    def _():
        o_imgf[...] = _dot(imf_ref[...], wi_ref[...]) + bi_ref[...]
        o_txtf[...] = _dot(tf_ref[...], wt_ref[...]) + bt_ref[...]

    @pl.when((i >= nb) & (i < 2 * nb))
    def _():
        o_img_uid[...] = _dot(g_img_ui_ref[...], item_emb_ref[...])

    @pl.when((i >= 2 * nb) & (i < 3 * nb))
    def _():
        o_txt_uid[...] = _dot(g_txt_ui_ref[...], item_emb_ref[...])

    @pl.when((i >= 3 * nb) & (i < 4 * nb))
    def _():
        o_img_iid[...] = _dot(g_img_iu_ref[...], user_emb_ref[...])

    @pl.when(i >= 4 * nb)
    def _():
        o_txt_iid[...] = _dot(g_txt_iu_ref[...], user_emb_ref[...])


def _seg_map(s, nb):
    return lambda i: (jnp.clip(i - s * nb, 0, nb - 1), 0)


def _mk1(image_feats, w_image_trans, b_image_trans, text_feats, w_text_trans,
         b_text_trans, g_img_ui, g_txt_ui, g_img_iu, g_txt_iu,
         item_id_emb, user_id_emb, bm=_MK1_BM):
    m = g_img_ui.shape[0]
    kf = image_feats.shape[1]
    kt = text_feats.shape[1]
    c = _EMBED
    nb = m // bm
    const = lambda shape: pl.BlockSpec(shape, lambda i: (0, 0))
    blk_out = lambda s: pl.BlockSpec((bm, c), _seg_map(s, nb))
    out_sds = jax.ShapeDtypeStruct((m, c), _F32)
    return pl.pallas_call(
        _mk1_body,
        grid=(5 * nb,),
        in_specs=[pl.BlockSpec((bm, kf), _seg_map(0, nb)),
                  const((kf, c)), const((1, c)),
                  pl.BlockSpec((bm, kt), _seg_map(0, nb)),
                  const((kt, c)), const((1, c)),
                  pl.BlockSpec((bm, m), _seg_map(1, nb)),
                  pl.BlockSpec((bm, m), _seg_map(2, nb)),
                  pl.BlockSpec((bm, m), _seg_map(3, nb)),
                  pl.BlockSpec((bm, m), _seg_map(4, nb)),
                  const((m, c)), const((m, c))],
        out_specs=[blk_out(0), blk_out(0), blk_out(1), blk_out(2),
                   blk_out(3), blk_out(4)],
        out_shape=[out_sds] * 6,
        compiler_params=pltpu.CompilerParams(
            dimension_semantics=("arbitrary",)),
    )(image_feats, w_image_trans, b_image_trans.reshape(1, c),
      text_feats, w_text_trans, b_text_trans.reshape(1, c),
      g_img_ui, g_txt_ui, g_img_iu, g_txt_iu, item_id_emb, user_id_emb)


def _pass_u_body(g_ref, imf_ref, tf_ref, iid_ref, tid_ref, iemb_ref, wsum_ref,
                 uid_ref, tuid_ref, uemb_ref,
                 ouf_ref, otf_ref, oug1_ref, oug0_ref, oig0_ref, ig0_scr):
    i = pl.program_id(0)
    bm = g_ref.shape[0]

    @pl.when(i == 0)
    def _():
        ig0_scr[...] = _id_update(iemb_ref[...], iid_ref[...], tid_ref[...],
                                  wsum_ref[...])

    g = g_ref[...]
    ouf_ref[...] = _dot(g, imf_ref[...])
    otf_ref[...] = _dot(g, tf_ref[...])
    oug1_ref[...] = _dot(g, ig0_scr[...])
    oug0_ref[...] = _id_update(uemb_ref[...], uid_ref[...], tuid_ref[...],
                               wsum_ref[...])
    oig0_ref[...] = ig0_scr[pl.ds(i * bm, bm), :]


def _pass_u(ui, image_f, text_f, image_item_id, text_item_id, item_id_emb,
            w_sum, image_user_id, text_user_id, user_id_emb, bm=_BM):
    m, k = ui.shape
    c = _EMBED
    blk = pl.BlockSpec((bm, c), lambda i: (i, 0))
    full = pl.BlockSpec((k, c), lambda i: (0, 0))
    out_sds = jax.ShapeDtypeStruct((m, c), _F32)
    return pl.pallas_call(
        _pass_u_body,
        grid=(m // bm,),
        in_specs=[pl.BlockSpec((bm, k), lambda i: (i, 0)),
                  full, full, full, full, full,
                  pl.BlockSpec((c, c), lambda i: (0, 0)),
                  blk, blk, blk],
        out_specs=[blk] * 5,
        out_shape=[out_sds] * 5,
        scratch_shapes=[pltpu.VMEM((k, c), _F32)],
        compiler_params=pltpu.CompilerParams(
            dimension_semantics=("arbitrary",)),
    )(ui, image_f, text_f, image_item_id, text_item_id, item_id_emb, w_sum,
      image_user_id, text_user_id, user_id_emb)


def _pass_i_body(g_ref, x1_ref, x2_ref, x3_ref, o1_ref, o2_ref, o3_ref):
    g = g_ref[...]
    o1_ref[...] = _dot(g, x1_ref[...])
    o2_ref[...] = _dot(g, x2_ref[...])
    o3_ref[...] = _dot(g, x3_ref[...])


def _pass_i(iu, x1, x2, x3, bm=_BM):
    m, k = iu.shape
    c = _EMBED
    blk = pl.BlockSpec((bm, c), lambda i: (i, 0))
    full = pl.BlockSpec((k, c), lambda i: (0, 0))
    out_sds = jax.ShapeDtypeStruct((m, c), _F32)
    return pl.pallas_call(
        _pass_i_body,
        grid=(m // bm,),
        in_specs=[pl.BlockSpec((bm, k), lambda i: (i, 0)), full, full, full],
        out_specs=[blk] * 3,
        out_shape=[out_sds] * 3,
        compiler_params=pltpu.CompilerParams(
            dimension_semantics=("arbitrary",)),
    )(iu, x1, x2, x3)


def _final(g0, g1, g2, fa, fb):
    mean_g = (g0 + g1 + g2) * (1.0 / 3.0)
    return (mean_g + _MODEL_CAT_RATE * _row_normalize(fa)
            + _MODEL_CAT_RATE * _row_normalize(fb))


def _pass_us_body(g_ref, ig1_ref, ug0_ref, ug1_ref, fu1_ref, fu2_ref,
                  oug2_ref, oug_ref):
    sm = jax.nn.softmax(_dot(g_ref[...], ig1_ref[...]), axis=-1)
    oug2_ref[...] = sm
    oug_ref[...] = _final(ug0_ref[...], ug1_ref[...], sm,
                          fu1_ref[...], fu2_ref[...])


def _pass_us(ui, i_g1, u_g0, u_g1, fu1, fu2, bm=_BM):
    m, k = ui.shape
    c = _EMBED
    blk = pl.BlockSpec((bm, c), lambda i: (i, 0))
    full = pl.BlockSpec((k, c), lambda i: (0, 0))
    out_sds = jax.ShapeDtypeStruct((m, c), _F32)
    return pl.pallas_call(
        _pass_us_body,
        grid=(m // bm,),
        in_specs=[pl.BlockSpec((bm, k), lambda i: (i, 0)),
                  full, blk, blk, blk, blk],
        out_specs=[blk, blk],
        out_shape=[out_sds, out_sds],
        compiler_params=pltpu.CompilerParams(
            dimension_semantics=("arbitrary",)),
    )(ui, i_g1, u_g0, u_g1, fu1, fu2)


def _pass_is_body(g_ref, ug2_ref, ig0_ref, ig1_ref, fi1_ref, fi2_ref,
                  oig_ref):
    sm = jax.nn.softmax(_dot(g_ref[...], ug2_ref[...]), axis=-1)
    oig_ref[...] = _final(ig0_ref[...], ig1_ref[...], sm,
                          fi1_ref[...], fi2_ref[...])


def _pass_is(iu, u_g2, i_g0, i_g1, fi1, fi2, bm=_BM):
    m, k = iu.shape
    c = _EMBED
    blk = pl.BlockSpec((bm, c), lambda i: (i, 0))
    full = pl.BlockSpec((k, c), lambda i: (0, 0))
    return pl.pallas_call(
        _pass_is_body,
        grid=(m // bm,),
        in_specs=[pl.BlockSpec((bm, k), lambda i: (i, 0)),
                  full, blk, blk, blk, blk],
        out_specs=blk,
        out_shape=jax.ShapeDtypeStruct((m, c), _F32),
        compiler_params=pltpu.CompilerParams(
            dimension_semantics=("arbitrary",)),
    )(iu, u_g2, i_g0, i_g1, fi1, fi2)


def kernel(ui_graph, iu_graph, image_ui_graph, image_iu_graph, text_ui_graph,
           text_iu_graph, image_feats, text_feats, w_image_trans, b_image_trans,
           w_text_trans, b_text_trans, user_id_emb, item_id_emb, w_q, w_k, w_cat):
    (image_f, text_f, image_user_id, text_user_id, image_item_id,
     text_item_id) = _mk1(image_feats, w_image_trans, b_image_trans,
                          text_feats, w_text_trans, b_text_trans,
                          image_ui_graph, text_ui_graph, image_iu_graph,
                          text_iu_graph, item_id_emb, user_id_emb)

    w_sum = w_cat.reshape(_HEADS, _EMBED, _EMBED).sum(0)

    (image_user_feats, text_user_feats, u_g1, u_g0, i_g0) = _pass_u(
        ui_graph, image_f, text_f, image_item_id, text_item_id, item_id_emb,
        w_sum, image_user_id, text_user_id, user_id_emb)

    image_item_feats, text_item_feats, i_g1 = _pass_i(
        iu_graph, image_user_feats, text_user_feats, u_g1)

    u_g2, u_g = _pass_us(ui_graph, i_g1, u_g0, u_g1,
                         image_user_feats, text_user_feats)
    i_g = _pass_is(iu_graph, u_g2, i_g0, i_g1,
                   image_item_feats, text_item_feats)

    return (u_g, i_g, image_item_feats, text_item_feats, image_user_feats,
            text_user_feats, u_g, i_g, image_user_id, text_user_id,
            image_item_id, text_item_id)


# speedup vs baseline: 1.0945x; 1.0945x over previous
import functools

import jax
import jax.numpy as jnp
from jax.experimental import pallas as pl
from jax.experimental.pallas import tpu as pltpu

_EMBED = 64
_HEADS = 4
_MODEL_CAT_RATE = 0.02
_ID_CAT_RATE = 0.36
_BM = 512
_F32 = jnp.float32


def _dot(a, b):
    return jnp.dot(a, b, preferred_element_type=_F32)


def _row_normalize(z):
    n = jnp.sqrt(jnp.sum(z * z, axis=1, keepdims=True))
    return z / jnp.maximum(n, 1e-12)


def _id_update(emb, a, b, w_sum):
    return emb + _ID_CAT_RATE * _row_normalize(_dot(0.5 * (a + b), w_sum))


_MK1_BM = 256


def _mk1_body(imf_ref, wi_ref, bi_ref, tf_ref, wt_ref, bt_ref,
              g_img_ui_ref, g_txt_ui_ref, g_img_iu_ref, g_txt_iu_ref,
              item_emb_ref, user_emb_ref,
              o_imgf, o_txtf, o_img_uid, o_txt_uid, o_img_iid, o_txt_iid):
    o_imgf[...] = _dot(imf_ref[...], wi_ref[...]) + bi_ref[...]
    o_txtf[...] = _dot(tf_ref[...], wt_ref[...]) + bt_ref[...]
    o_img_uid[...] = _dot(g_img_ui_ref[...], item_emb_ref[...])
    o_txt_uid[...] = _dot(g_txt_ui_ref[...], item_emb_ref[...])
    o_img_iid[...] = _dot(g_img_iu_ref[...], user_emb_ref[...])
    o_txt_iid[...] = _dot(g_txt_iu_ref[...], user_emb_ref[...])


def _mk1(image_feats, w_image_trans, b_image_trans, text_feats, w_text_trans,
         b_text_trans, g_img_ui, g_txt_ui, g_img_iu, g_txt_iu,
         item_id_emb, user_id_emb, bm=_MK1_BM):
    m = g_img_ui.shape[0]
    kf = image_feats.shape[1]
    kt = text_feats.shape[1]
    c = _EMBED
    row = lambda k: pl.BlockSpec((bm, k), lambda i: (i, 0))
    const = lambda shape: pl.BlockSpec(shape, lambda i: (0, 0))
    blk = pl.BlockSpec((bm, c), lambda i: (i, 0))
    out_sds = jax.ShapeDtypeStruct((m, c), _F32)
    return pl.pallas_call(
        _mk1_body,
        grid=(m // bm,),
        in_specs=[row(kf), const((kf, c)), const((1, c)),
                  row(kt), const((kt, c)), const((1, c)),
                  row(m), row(m), row(m), row(m),
                  const((m, c)), const((m, c))],
        out_specs=[blk] * 6,
        out_shape=[out_sds] * 6,
        compiler_params=pltpu.CompilerParams(
            dimension_semantics=("arbitrary",)),
    )(image_feats, w_image_trans, b_image_trans.reshape(1, c),
      text_feats, w_text_trans, b_text_trans.reshape(1, c),
      g_img_ui, g_txt_ui, g_img_iu, g_txt_iu, item_id_emb, user_id_emb)


def _pass_u_body(g_ref, imf_ref, tf_ref, iid_ref, tid_ref, iemb_ref, wsum_ref,
                 uid_ref, tuid_ref, uemb_ref,
                 ouf_ref, otf_ref, oug1_ref, oug0_ref, oig0_ref, ig0_scr):
    i = pl.program_id(0)
    bm = g_ref.shape[0]

    @pl.when(i == 0)
    def _():
        ig0_scr[...] = _id_update(iemb_ref[...], iid_ref[...], tid_ref[...],
                                  wsum_ref[...])

    g = g_ref[...]
    ouf_ref[...] = _dot(g, imf_ref[...])
    otf_ref[...] = _dot(g, tf_ref[...])
    oug1_ref[...] = _dot(g, ig0_scr[...])
    oug0_ref[...] = _id_update(uemb_ref[...], uid_ref[...], tuid_ref[...],
                               wsum_ref[...])
    oig0_ref[...] = ig0_scr[pl.ds(i * bm, bm), :]


def _pass_u(ui, image_f, text_f, image_item_id, text_item_id, item_id_emb,
            w_sum, image_user_id, text_user_id, user_id_emb, bm=_BM):
    m, k = ui.shape
    c = _EMBED
    blk = pl.BlockSpec((bm, c), lambda i: (i, 0))
    full = pl.BlockSpec((k, c), lambda i: (0, 0))
    out_sds = jax.ShapeDtypeStruct((m, c), _F32)
    return pl.pallas_call(
        _pass_u_body,
        grid=(m // bm,),
        in_specs=[pl.BlockSpec((bm, k), lambda i: (i, 0)),
                  full, full, full, full, full,
                  pl.BlockSpec((c, c), lambda i: (0, 0)),
                  blk, blk, blk],
        out_specs=[blk] * 5,
        out_shape=[out_sds] * 5,
        scratch_shapes=[pltpu.VMEM((k, c), _F32)],
        compiler_params=pltpu.CompilerParams(
            dimension_semantics=("arbitrary",)),
    )(ui, image_f, text_f, image_item_id, text_item_id, item_id_emb, w_sum,
      image_user_id, text_user_id, user_id_emb)


def _pass_i_body(g_ref, x1_ref, x2_ref, x3_ref, o1_ref, o2_ref, o3_ref):
    g = g_ref[...]
    o1_ref[...] = _dot(g, x1_ref[...])
    o2_ref[...] = _dot(g, x2_ref[...])
    o3_ref[...] = _dot(g, x3_ref[...])


def _pass_i(iu, x1, x2, x3, bm=_BM):
    m, k = iu.shape
    c = _EMBED
    blk = pl.BlockSpec((bm, c), lambda i: (i, 0))
    full = pl.BlockSpec((k, c), lambda i: (0, 0))
    out_sds = jax.ShapeDtypeStruct((m, c), _F32)
    return pl.pallas_call(
        _pass_i_body,
        grid=(m // bm,),
        in_specs=[pl.BlockSpec((bm, k), lambda i: (i, 0)), full, full, full],
        out_specs=[blk] * 3,
        out_shape=[out_sds] * 3,
        compiler_params=pltpu.CompilerParams(
            dimension_semantics=("arbitrary",)),
    )(iu, x1, x2, x3)


def _final(g0, g1, g2, fa, fb):
    mean_g = (g0 + g1 + g2) * (1.0 / 3.0)
    return (mean_g + _MODEL_CAT_RATE * _row_normalize(fa)
            + _MODEL_CAT_RATE * _row_normalize(fb))


def _pass_us_body(g_ref, ig1_ref, ug0_ref, ug1_ref, fu1_ref, fu2_ref,
                  oug2_ref, oug_ref):
    sm = jax.nn.softmax(_dot(g_ref[...], ig1_ref[...]), axis=-1)
    oug2_ref[...] = sm
    oug_ref[...] = _final(ug0_ref[...], ug1_ref[...], sm,
                          fu1_ref[...], fu2_ref[...])


def _pass_us(ui, i_g1, u_g0, u_g1, fu1, fu2, bm=_BM):
    m, k = ui.shape
    c = _EMBED
    blk = pl.BlockSpec((bm, c), lambda i: (i, 0))
    full = pl.BlockSpec((k, c), lambda i: (0, 0))
    out_sds = jax.ShapeDtypeStruct((m, c), _F32)
    return pl.pallas_call(
        _pass_us_body,
        grid=(m // bm,),
        in_specs=[pl.BlockSpec((bm, k), lambda i: (i, 0)),
                  full, blk, blk, blk, blk],
        out_specs=[blk, blk],
        out_shape=[out_sds, out_sds],
        compiler_params=pltpu.CompilerParams(
            dimension_semantics=("arbitrary",)),
    )(ui, i_g1, u_g0, u_g1, fu1, fu2)


def _pass_is_body(g_ref, ug2_ref, ig0_ref, ig1_ref, fi1_ref, fi2_ref,
                  oig_ref):
    sm = jax.nn.softmax(_dot(g_ref[...], ug2_ref[...]), axis=-1)
    oig_ref[...] = _final(ig0_ref[...], ig1_ref[...], sm,
                          fi1_ref[...], fi2_ref[...])


def _pass_is(iu, u_g2, i_g0, i_g1, fi1, fi2, bm=_BM):
    m, k = iu.shape
    c = _EMBED
    blk = pl.BlockSpec((bm, c), lambda i: (i, 0))
    full = pl.BlockSpec((k, c), lambda i: (0, 0))
    return pl.pallas_call(
        _pass_is_body,
        grid=(m // bm,),
        in_specs=[pl.BlockSpec((bm, k), lambda i: (i, 0)),
                  full, blk, blk, blk, blk],
        out_specs=blk,
        out_shape=jax.ShapeDtypeStruct((m, c), _F32),
        compiler_params=pltpu.CompilerParams(
            dimension_semantics=("arbitrary",)),
    )(iu, u_g2, i_g0, i_g1, fi1, fi2)


def kernel(ui_graph, iu_graph, image_ui_graph, image_iu_graph, text_ui_graph,
           text_iu_graph, image_feats, text_feats, w_image_trans, b_image_trans,
           w_text_trans, b_text_trans, user_id_emb, item_id_emb, w_q, w_k, w_cat):
    (image_f, text_f, image_user_id, text_user_id, image_item_id,
     text_item_id) = _mk1(image_feats, w_image_trans, b_image_trans,
                          text_feats, w_text_trans, b_text_trans,
                          image_ui_graph, text_ui_graph, image_iu_graph,
                          text_iu_graph, item_id_emb, user_id_emb)

    w_sum = w_cat.reshape(_HEADS, _EMBED, _EMBED).sum(0)

    (image_user_feats, text_user_feats, u_g1, u_g0, i_g0) = _pass_u(
        ui_graph, image_f, text_f, image_item_id, text_item_id, item_id_emb,
        w_sum, image_user_id, text_user_id, user_id_emb)

    image_item_feats, text_item_feats, i_g1 = _pass_i(
        iu_graph, image_user_feats, text_user_feats, u_g1)

    u_g2, u_g = _pass_us(ui_graph, i_g1, u_g0, u_g1,
                         image_user_feats, text_user_feats)
    i_g = _pass_is(iu_graph, u_g2, i_g0, i_g1,
                   image_item_feats, text_item_feats)

    return (u_g, i_g, image_item_feats, text_item_feats, image_user_feats,
            text_user_feats, u_g, i_g, image_user_id, text_user_id,
            image_item_id, text_item_id)


# speedup vs baseline: 1.1600x; 1.0599x over previous
import jax
import jax.numpy as jnp
from jax.experimental import pallas as pl
from jax.experimental.pallas import tpu as pltpu

_EMBED = 64
_HEADS = 4
_MODEL_CAT_RATE = 0.02
_ID_CAT_RATE = 0.36
_BM = 512
_F32 = jnp.float32


def _dot(a, b):
    return jnp.dot(a, b, preferred_element_type=_F32)


def _row_normalize(z):
    n = jnp.sqrt(jnp.sum(z * z, axis=1, keepdims=True))
    return z / jnp.maximum(n, 1e-12)


_MK1_BM = 256


def _mk1_body(imf_ref, wi_ref, bi_ref, tf_ref, wt_ref, bt_ref,
              g_img_ui_ref, g_txt_ui_ref, g_img_iu_ref, g_txt_iu_ref,
              item_emb_ref, user_emb_ref,
              o_ft, o_idm, o_img_uid, o_txt_uid, o_img_iid, o_txt_iid):
    imgf = _dot(imf_ref[...], wi_ref[...]) + bi_ref[...]
    txtf = _dot(tf_ref[...], wt_ref[...]) + bt_ref[...]
    o_ft[...] = jnp.concatenate([imgf, txtf], axis=1)
    iuid = _dot(g_img_ui_ref[...], item_emb_ref[...])
    tuid = _dot(g_txt_ui_ref[...], item_emb_ref[...])
    iiid = _dot(g_img_iu_ref[...], user_emb_ref[...])
    tiid = _dot(g_txt_iu_ref[...], user_emb_ref[...])
    o_img_uid[...] = iuid
    o_txt_uid[...] = tuid
    o_img_iid[...] = iiid
    o_txt_iid[...] = tiid
    o_idm[...] = jnp.concatenate([0.5 * (iiid + tiid), 0.5 * (iuid + tuid)], 1)


def _mk1(image_feats, w_image_trans, b_image_trans, text_feats, w_text_trans,
         b_text_trans, g_img_ui, g_txt_ui, g_img_iu, g_txt_iu,
         item_id_emb, user_id_emb, bm=_MK1_BM):
    m = g_img_ui.shape[0]
    kf = image_feats.shape[1]
    kt = text_feats.shape[1]
    c = _EMBED
    row = lambda k: pl.BlockSpec((bm, k), lambda i: (i, 0))
    const = lambda shape: pl.BlockSpec(shape, lambda i: (0, 0))
    blk = lambda w: pl.BlockSpec((bm, w), lambda i: (i, 0))
    sds = lambda w: jax.ShapeDtypeStruct((m, w), _F32)
    return pl.pallas_call(
        _mk1_body,
        grid=(m // bm,),
        in_specs=[row(kf), const((kf, c)), const((1, c)),
                  row(kt), const((kt, c)), const((1, c)),
                  row(m), row(m), row(m), row(m),
                  const((m, c)), const((m, c))],
        out_specs=[blk(2 * c), blk(2 * c), blk(c), blk(c), blk(c), blk(c)],
        out_shape=[sds(2 * c), sds(2 * c), sds(c), sds(c), sds(c), sds(c)],
        compiler_params=pltpu.CompilerParams(
            dimension_semantics=("arbitrary",)),
    )(image_feats, w_image_trans, b_image_trans.reshape(1, c),
      text_feats, w_text_trans, b_text_trans.reshape(1, c),
      g_img_ui, g_txt_ui, g_img_iu, g_txt_iu, item_id_emb, user_id_emb)


def _mk2_body(ui_ref, iu_ref, ft_ref, idm_ref, iemb_ref, uemb_ref, wsum_ref,
              o_iuf, o_tuf, o_iif, o_tif, o_ug, o_ig,
              g0_s, uf_s, if_s, g1_s, ug2_s):
    i = pl.program_id(0)
    bmu = ui_ref.shape[0]
    bmi = iu_ref.shape[0]
    c = _EMBED
    nbu = ui_ref.shape[1] // bmu
    nbi = iu_ref.shape[1] // bmi
    b1, b2, b3 = nbu, nbu + nbi, 2 * nbu + nbi

    @pl.when(i == 0)
    def _():
        zi = _row_normalize(_dot(idm_ref[:, :c], wsum_ref[...]))
        zu = _row_normalize(_dot(idm_ref[:, c:], wsum_ref[...]))
        g0_s[...] = (jnp.concatenate([iemb_ref[...], uemb_ref[...]], axis=1)
                     + _ID_CAT_RATE * jnp.concatenate([zi, zu], axis=1))

    @pl.when(i < b1)
    def _():
        sl = pl.ds(i * bmu, bmu)
        g = ui_ref[...]
        uf = _dot(g, ft_ref[...])
        o_iuf[...] = uf[:, :c]
        o_tuf[...] = uf[:, c:]
        uf_s[sl, :] = uf
        g1_s[sl, :c] = _dot(g, g0_s[:, :c])

    @pl.when((i >= b1) & (i < b2))
    def _():
        sl = pl.ds((i - b1) * bmi, bmi)
        g = iu_ref[...]
        itf = _dot(g, uf_s[...])
        o_iif[...] = itf[:, :c]
        o_tif[...] = itf[:, c:]
        if_s[sl, :] = itf
        g1_s[sl, c:] = _dot(g, g1_s[:, :c])

    @pl.when((i >= b2) & (i < b3))
    def _():
        sl = pl.ds((i - b2) * bmu, bmu)
        sm = jax.nn.softmax(_dot(ui_ref[...], g1_s[:, c:]), axis=-1)
        ug2_s[sl, :] = sm
        o_ug[...] = _final(g0_s[sl, c:], g1_s[sl, :c], sm,
                           uf_s[sl, :c], uf_s[sl, c:])

    @pl.when(i >= b3)
    def _():
        sl = pl.ds((i - b3) * bmi, bmi)
        sm = jax.nn.softmax(_dot(iu_ref[...], ug2_s[...]), axis=-1)
        o_ig[...] = _final(g0_s[sl, :c], g1_s[sl, c:], sm,
                           if_s[sl, :c], if_s[sl, c:])


def _mk2(ui, iu, ft, idm, item_id_emb, user_id_emb, w_sum,
         bmu=_BM, bmi=_MK1_BM):
    m, k = ui.shape
    c = _EMBED
    nbu = m // bmu
    nbi = m // bmi
    b1, b2, b3 = nbu, nbu + nbi, 2 * nbu + nbi
    const = lambda w: pl.BlockSpec((m, w), lambda i: (0, 0))
    wblk = pl.BlockSpec((c, c), lambda i: (0, 0))

    def ui_map(i):
        return (jnp.where(i < b1, i,
                jnp.where(i < b2, 0,
                jnp.where(i < b3, i - b2, nbu - 1))), 0)

    def iu_map(i):
        return (jnp.where(i < b1, 0,
                jnp.where(i < b2, i - b1,
                jnp.where(i < b3, 0, i - b3))), 0)

    def oseg(s, bm, nb):
        return pl.BlockSpec((bm, c), lambda i: (jnp.clip(i - s, 0, nb - 1), 0))

    out_sds = jax.ShapeDtypeStruct((m, c), _F32)
    scr = lambda w: pltpu.VMEM((m, w), _F32)
    return pl.pallas_call(
        _mk2_body,
        grid=(b3 + nbi,),
        in_specs=[pl.BlockSpec((bmu, k), ui_map),
                  pl.BlockSpec((bmi, k), iu_map),
                  const(2 * c), const(2 * c), const(c), const(c), wblk],
        out_specs=[oseg(0, bmu, nbu), oseg(0, bmu, nbu),
                   oseg(b1, bmi, nbi), oseg(b1, bmi, nbi),
                   oseg(b2, bmu, nbu), oseg(b3, bmi, nbi)],
        out_shape=[out_sds] * 6,
        scratch_shapes=[scr(2 * c), scr(2 * c), scr(2 * c), scr(2 * c),
                        scr(c)],
        compiler_params=pltpu.CompilerParams(
            dimension_semantics=("arbitrary",)),
    )(ui, iu, ft, idm, item_id_emb, user_id_emb, w_sum)


def _final(g0, g1, g2, fa, fb):
    mean_g = (g0 + g1 + g2) * (1.0 / 3.0)
    return (mean_g + _MODEL_CAT_RATE * _row_normalize(fa)
            + _MODEL_CAT_RATE * _row_normalize(fb))


def kernel(ui_graph, iu_graph, image_ui_graph, image_iu_graph, text_ui_graph,
           text_iu_graph, image_feats, text_feats, w_image_trans, b_image_trans,
           w_text_trans, b_text_trans, user_id_emb, item_id_emb, w_q, w_k, w_cat):
    (ft, idm, image_user_id, text_user_id, image_item_id,
     text_item_id) = _mk1(image_feats, w_image_trans, b_image_trans,
                          text_feats, w_text_trans, b_text_trans,
                          image_ui_graph, text_ui_graph, image_iu_graph,
                          text_iu_graph, item_id_emb, user_id_emb)

    w_sum = w_cat.reshape(_HEADS, _EMBED, _EMBED).sum(0)

    (image_user_feats, text_user_feats, image_item_feats, text_item_feats,
     u_g, i_g) = _mk2(ui_graph, iu_graph, ft, idm,
                      item_id_emb, user_id_emb, w_sum)

    return (u_g, i_g, image_item_feats, text_item_feats, image_user_feats,
            text_user_feats, u_g, i_g, image_user_id, text_user_id,
            image_item_id, text_item_id)


# speedup vs baseline: 1.2124x; 1.0451x over previous
import jax
import jax.numpy as jnp
from jax.experimental import pallas as pl
from jax.experimental.pallas import tpu as pltpu

_EMBED = 64
_HEADS = 4
_MODEL_CAT_RATE = 0.02
_ID_CAT_RATE = 0.36
_BM = 512
_F32 = jnp.float32


def _dot(a, b):
    return jnp.dot(a, b, preferred_element_type=_F32)


def _row_normalize(z):
    n = jnp.sqrt(jnp.sum(z * z, axis=1, keepdims=True))
    return z / jnp.maximum(n, 1e-12)


_MK1_BM = 256


def _mk1_body(imf_ref, wi_ref, bi_ref, tf_ref, wt_ref, bt_ref,
              g_img_ui_ref, g_txt_ui_ref, g_img_iu_ref, g_txt_iu_ref,
              item_emb_ref, user_emb_ref,
              o_ft, o_idm, o_emb2, o_img_uid, o_txt_uid, o_img_iid,
              o_txt_iid):
    i = pl.program_id(0)
    bm = o_ft.shape[0]
    imgf = _dot(imf_ref[...], wi_ref[...]) + bi_ref[...]
    txtf = _dot(tf_ref[...], wt_ref[...]) + bt_ref[...]
    o_ft[...] = jnp.concatenate([imgf, txtf], axis=1)
    iuid = _dot(g_img_ui_ref[...], item_emb_ref[...])
    tuid = _dot(g_txt_ui_ref[...], item_emb_ref[...])
    iiid = _dot(g_img_iu_ref[...], user_emb_ref[...])
    tiid = _dot(g_txt_iu_ref[...], user_emb_ref[...])
    o_img_uid[...] = iuid
    o_txt_uid[...] = tuid
    o_img_iid[...] = iiid
    o_txt_iid[...] = tiid
    o_idm[...] = jnp.concatenate([0.5 * (iiid + tiid), 0.5 * (iuid + tuid)], 1)
    sl = pl.ds(i * bm, bm)
    o_emb2[...] = jnp.concatenate([item_emb_ref[sl, :], user_emb_ref[sl, :]],
                                  axis=1)


def _mk1(image_feats, w_image_trans, b_image_trans, text_feats, w_text_trans,
         b_text_trans, g_img_ui, g_txt_ui, g_img_iu, g_txt_iu,
         item_id_emb, user_id_emb, bm=_MK1_BM):
    m = g_img_ui.shape[0]
    kf = image_feats.shape[1]
    kt = text_feats.shape[1]
    c = _EMBED
    row = lambda k: pl.BlockSpec((bm, k), lambda i: (i, 0))
    const = lambda shape: pl.BlockSpec(shape, lambda i: (0, 0))
    blk = lambda w: pl.BlockSpec((bm, w), lambda i: (i, 0))
    sds = lambda w: jax.ShapeDtypeStruct((m, w), _F32)
    return pl.pallas_call(
        _mk1_body,
        grid=(m // bm,),
        in_specs=[row(kf), const((kf, c)), const((1, c)),
                  row(kt), const((kt, c)), const((1, c)),
                  row(m), row(m), row(m), row(m),
                  const((m, c)), const((m, c))],
        out_specs=[blk(2 * c), blk(2 * c), blk(2 * c),
                   blk(c), blk(c), blk(c), blk(c)],
        out_shape=[sds(2 * c), sds(2 * c), sds(2 * c),
                   sds(c), sds(c), sds(c), sds(c)],
        compiler_params=pltpu.CompilerParams(
            dimension_semantics=("arbitrary",)),
    )(image_feats, w_image_trans, b_image_trans.reshape(1, c),
      text_feats, w_text_trans, b_text_trans.reshape(1, c),
      g_img_ui, g_txt_ui, g_img_iu, g_txt_iu, item_id_emb, user_id_emb)


def _mk2_body(ui_ref, iu_ref, ft_ref, idm_ref, emb2_ref, wsum_ref,
              o_iuf, o_tuf, o_iif, o_tif, o_ug, o_ig,
              g0_s, uf_s, if_s, g1_s, ug2_s):
    i = pl.program_id(0)
    bmu = ui_ref.shape[0]
    bmi = iu_ref.shape[0]
    c = _EMBED
    nbu = ui_ref.shape[1] // bmu
    nbi = iu_ref.shape[1] // bmi
    b1, b2, b3 = nbu, nbu + nbi, 2 * nbu + nbi

    @pl.when(i == 0)
    def _():
        zi = _row_normalize(_dot(idm_ref[:, :c], wsum_ref[...]))
        zu = _row_normalize(_dot(idm_ref[:, c:], wsum_ref[...]))
        g0_s[...] = (emb2_ref[...]
                     + _ID_CAT_RATE * jnp.concatenate([zi, zu], axis=1))

    @pl.when(i < b1)
    def _():
        sl = pl.ds(i * bmu, bmu)
        g = ui_ref[...]
        uf = _dot(g, ft_ref[...])
        o_iuf[...] = uf[:, :c]
        o_tuf[...] = uf[:, c:]
        uf_s[sl, :] = uf.astype(jnp.bfloat16)
        g1_s[sl, :c] = _dot(g, g0_s[:, :c])

    @pl.when((i >= b1) & (i < b2))
    def _():
        sl = pl.ds((i - b1) * bmi, bmi)
        g = iu_ref[...]
        itf = _dot(g, uf_s[...])
        o_iif[...] = itf[:, :c]
        o_tif[...] = itf[:, c:]
        if_s[sl, :] = itf.astype(jnp.bfloat16)
        g1_s[sl, c:] = _dot(g, g1_s[:, :c])

    @pl.when((i >= b2) & (i < b3))
    def _():
        sl = pl.ds((i - b2) * bmu, bmu)
        sm = jax.nn.softmax(_dot(ui_ref[...], g1_s[:, c:]), axis=-1)
        ug2_s[sl, :] = sm.astype(jnp.bfloat16)
        o_ug[...] = _final(g0_s[sl, c:], g1_s[sl, :c], sm,
                           uf_s[sl, :c].astype(_F32),
                           uf_s[sl, c:].astype(_F32))

    @pl.when(i >= b3)
    def _():
        sl = pl.ds((i - b3) * bmi, bmi)
        sm = jax.nn.softmax(_dot(iu_ref[...], ug2_s[...]), axis=-1)
        o_ig[...] = _final(g0_s[sl, :c], g1_s[sl, c:], sm,
                           if_s[sl, :c].astype(_F32),
                           if_s[sl, c:].astype(_F32))


def _mk2(ui, iu, ft, idm, emb2, w_sum, bmu=_BM, bmi=_BM):
    m, k = ui.shape
    c = _EMBED
    nbu = m // bmu
    nbi = m // bmi
    b1, b2, b3 = nbu, nbu + nbi, 2 * nbu + nbi
    const = lambda w: pl.BlockSpec((m, w), lambda i: (0, 0))
    wblk = pl.BlockSpec((c, c), lambda i: (0, 0))

    def ui_map(i):
        return (jnp.where(i < b1, i,
                jnp.where(i < b2, 0,
                jnp.where(i < b3, i - b2, nbu - 1))), 0)

    def iu_map(i):
        return (jnp.where(i < b1, 0,
                jnp.where(i < b2, i - b1,
                jnp.where(i < b3, 0, i - b3))), 0)

    def oseg(s, bm, nb):
        return pl.BlockSpec((bm, c), lambda i: (jnp.clip(i - s, 0, nb - 1), 0))

    out_sds = jax.ShapeDtypeStruct((m, c), _F32)
    scr = lambda w, dt=_F32: pltpu.VMEM((m, w), dt)
    return pl.pallas_call(
        _mk2_body,
        grid=(b3 + nbi,),
        in_specs=[pl.BlockSpec((bmu, k), ui_map),
                  pl.BlockSpec((bmi, k), iu_map),
                  const(2 * c), const(2 * c), const(2 * c), wblk],
        out_specs=[oseg(0, bmu, nbu), oseg(0, bmu, nbu),
                   oseg(b1, bmi, nbi), oseg(b1, bmi, nbi),
                   oseg(b2, bmu, nbu), oseg(b3, bmi, nbi)],
        out_shape=[out_sds] * 6,
        scratch_shapes=[scr(2 * c), scr(2 * c, jnp.bfloat16),
                        scr(2 * c, jnp.bfloat16), scr(2 * c),
                        scr(c, jnp.bfloat16)],
        compiler_params=pltpu.CompilerParams(
            dimension_semantics=("arbitrary",)),
    )(ui, iu, ft, idm, emb2, w_sum)


def _final(g0, g1, g2, fa, fb):
    mean_g = (g0 + g1 + g2) * (1.0 / 3.0)
    return (mean_g + _MODEL_CAT_RATE * _row_normalize(fa)
            + _MODEL_CAT_RATE * _row_normalize(fb))


def kernel(ui_graph, iu_graph, image_ui_graph, image_iu_graph, text_ui_graph,
           text_iu_graph, image_feats, text_feats, w_image_trans, b_image_trans,
           w_text_trans, b_text_trans, user_id_emb, item_id_emb, w_q, w_k, w_cat):
    (ft, idm, emb2, image_user_id, text_user_id, image_item_id,
     text_item_id) = _mk1(image_feats, w_image_trans, b_image_trans,
                          text_feats, w_text_trans, b_text_trans,
                          image_ui_graph, text_ui_graph, image_iu_graph,
                          text_iu_graph, item_id_emb, user_id_emb)

    w_sum = w_cat.reshape(_HEADS, _EMBED, _EMBED).sum(0)

    (image_user_feats, text_user_feats, image_item_feats, text_item_feats,
     u_g, i_g) = _mk2(ui_graph, iu_graph, ft, idm, emb2, w_sum)

    return (u_g, i_g, image_item_feats, text_item_feats, image_user_feats,
            text_user_feats, u_g, i_g, image_user_id, text_user_id,
            image_item_id, text_item_id)
